# Initial kernel scaffold; baseline (speedup 1.0000x reference)
import jax
import jax.numpy as jnp
from jax.experimental import pallas as pl


def kernel(pts, p2v_idx, center_points, center2corner, voxel_size, embeddings):
    raise NotImplementedError("write your pallas kernel here")


# speedup vs baseline: 3.1283x; 3.1283x over previous
import functools

import jax
import jax.numpy as jnp
from jax import lax
from jax.experimental import pallas as pl
from jax.experimental.pallas import tpu as pltpu
from jax.experimental.pallas import tpu_sc as plsc

_L = 16
_NC = 2
_NS = 16
_NW = _NC * _NS
_B = 64
_DIM = 64
_IDX_CHUNK = 128

_BCAST_DNUMS = lax.GatherDimensionNumbers(
    offset_dims=(), collapsed_slice_dims=(0,), start_index_map=(0,))


def _bcast(vec, i):
    sel = jnp.full((_L, 1), i, jnp.int32)
    return lax.gather(vec, sel, _BCAST_DNUMS, slice_sizes=(1,),
                      mode=lax.GatherScatterMode.PROMISE_IN_BOUNDS)


@functools.lru_cache(maxsize=None)
def _build(nb: int):
    npad = _NW * nb * _B
    per_w = nb * _B

    def body(px_h, py_h, pz_h, p2v_h, cx_h, cy_h, cz_h, c2c_h, inv_h, emb_h,
             out_h,
             px_v, py_v, pz_v, p2v_v, cx_v, cy_v, cz_v, widx_v, eidx_v,
             emb_v, out_v, inv_v, sem_a, sem_b):
        wid = lax.axis_index("s") * _NC + lax.axis_index("c")
        pltpu.sync_copy(inv_h, inv_v)
        inv = inv_v[...]
        iota = lax.iota(jnp.int32, _L)

        def batch(b, carry):
            base = wid * per_w + b * _B
            pltpu.sync_copy(p2v_h.at[pl.ds(base, _B)], p2v_v)
            pltpu.sync_copy(px_h.at[pl.ds(base, _B)], px_v)
            pltpu.sync_copy(py_h.at[pl.ds(base, _B)], py_v)
            pltpu.sync_copy(pz_h.at[pl.ds(base, _B)], pz_v)
            ctr_copies = [
                pltpu.async_copy(cx_h.at[p2v_v], cx_v, sem_a),
                pltpu.async_copy(cy_h.at[p2v_v], cy_v, sem_a),
                pltpu.async_copy(cz_h.at[p2v_v], cz_v, sem_a),
            ]
            for g in range(_B // _L):
                pv8 = p2v_v[pl.ds(g * _L, _L)] * 8
                dst0 = iota * 8 + g * _L * 8
                for kk in range(8):
                    plsc.store_scatter(eidx_v, [dst0 + kk], pv8 + kk)
            idx_copies = []
            for j in range(8 * _B // _IDX_CHUNK):
                sl = pl.ds(j * _IDX_CHUNK, _IDX_CHUNK)
                idx_copies.append(pltpu.async_copy(
                    c2c_h.at[eidx_v.at[sl]], widx_v.at[sl], sem_b))
            for c in idx_copies:
                c.wait()
            emb_copies = []
            for j in range(8 * _B // _IDX_CHUNK):
                sl = pl.ds(j * _IDX_CHUNK, _IDX_CHUNK)
                emb_copies.append(pltpu.async_copy(
                    emb_h.at[widx_v.at[sl]],
                    emb_v.at[pl.ds(j * _IDX_CHUNK, _IDX_CHUNK)], sem_b))
            for c in ctr_copies:
                c.wait()
            for c in emb_copies:
                c.wait()
            for g in range(_B // _L):
                sl = pl.ds(g * _L, _L)
                rel = [(px_v[sl] - cx_v[sl]) * inv + 0.5,
                       (py_v[sl] - cy_v[sl]) * inv + 0.5,
                       (pz_v[sl] - cz_v[sl]) * inv + 0.5]
                w = []
                for kk in range(8):
                    f0 = rel[0] if (kk >> 2) & 1 else 1.0 - rel[0]
                    f1 = rel[1] if (kk >> 1) & 1 else 1.0 - rel[1]
                    f2 = rel[2] if kk & 1 else 1.0 - rel[2]
                    w.append(f0 * f1 * f2)
                for i in range(_L):
                    pt = g * _L + i
                    wb = [_bcast(w[kk], i) for kk in range(8)]
                    for c4 in range(_DIM // _L):
                        dsl = pl.ds(c4 * _L, _L)
                        acc = wb[0] * emb_v[pt * 8, dsl]
                        for kk in range(1, 8):
                            acc = acc + wb[kk] * emb_v[pt * 8 + kk, dsl]
                        out_v[pt, dsl] = acc
            pltpu.sync_copy(out_v, out_h.at[pl.ds(base, _B)])
            return carry

        lax.fori_loop(0, nb, batch, 0)

    mesh = plsc.VectorSubcoreMesh(core_axis_name="c", subcore_axis_name="s")
    return pl.kernel(
        body,
        out_type=jax.ShapeDtypeStruct((npad, _DIM), jnp.float32),
        mesh=mesh,
        compiler_params=pltpu.CompilerParams(needs_layout_passes=False,
                                             use_tc_tiling_on_sc=False),
        scratch_types=[
            pltpu.VMEM((_B,), jnp.float32),
            pltpu.VMEM((_B,), jnp.float32),
            pltpu.VMEM((_B,), jnp.float32),
            pltpu.VMEM((_B,), jnp.int32),
            pltpu.VMEM((_B,), jnp.float32),
            pltpu.VMEM((_B,), jnp.float32),
            pltpu.VMEM((_B,), jnp.float32),
            pltpu.VMEM((8 * _B,), jnp.int32),
            pltpu.VMEM((8 * _B,), jnp.int32),
            pltpu.VMEM((8 * _B, _DIM), jnp.float32),
            pltpu.VMEM((_B, _DIM), jnp.float32),
            pltpu.VMEM((_L,), jnp.float32),
            pltpu.SemaphoreType.DMA,
            pltpu.SemaphoreType.DMA,
        ],
    )


def kernel(pts, p2v_idx, center_points, center2corner, voxel_size, embeddings):
    n = pts.shape[0]
    chunk = _NW * _B
    nb = -(-n // chunk)
    npad = nb * chunk
    pts_p = jnp.pad(pts, ((0, npad - n), (0, 0)))
    pts_t = pts_p.T
    px, py, pz = pts_t[0], pts_t[1], pts_t[2]
    p2v_p = jnp.pad(p2v_idx.astype(jnp.int32), (0, npad - n))
    ctr_t = center_points.T
    cx, cy, cz = ctr_t[0], ctr_t[1], ctr_t[2]
    c2c_flat = center2corner.astype(jnp.int32).reshape(-1)
    inv = jnp.broadcast_to(1.0 / voxel_size[0], (_L,)).astype(jnp.float32)
    out = _build(nb)(
        px, py, pz, p2v_p, cx, cy, cz, c2c_flat, inv, embeddings)
    return out[:n]
